# Initial kernel scaffold; baseline (speedup 1.0000x reference)
import jax
import jax.numpy as jnp
from jax.experimental import pallas as pl


def kernel(x, edge_index, batch_idx, W1, b1, prelu_a, W2, b2):
    raise NotImplementedError("write your pallas kernel here")


# speedup vs baseline: 49.2106x; 49.2106x over previous
import jax
import jax.numpy as jnp
from jax import lax
from jax.experimental import pallas as pl
from jax.experimental.pallas import tpu as pltpu
from jax.experimental.pallas import tpu_sc as plsc

_N = 10000
_E = 320000
_D = 128
_H = 16
_G = 16
_O = 2

_NC = 2
_NS = 16
_NW = _NC * _NS
_EPW = _E // _NW
_CH = 1000
_NCHUNK = _EPW // _CH
_NPAD = 10240
_RPT = _NPAD // _NS

_sc_mesh = plsc.VectorSubcoreMesh(core_axis_name="c", subcore_axis_name="s")
_sc_params = pltpu.CompilerParams(use_tc_tiling_on_sc=False)


def _deg_body(dst_hbm, ones_hbm, zeros_hbm, out_hbm, idx_v, ones_v, acc, sem):
    del sem
    c = lax.axis_index("c")
    s = lax.axis_index("s")
    w = s * _NC + c
    pltpu.sync_copy(zeros_hbm.at[pl.ds(s * _RPT, _RPT)],
                    acc.at[pl.ds(s * _RPT, _RPT)])
    pltpu.sync_copy(ones_hbm.at[pl.ds(0, _CH)], ones_v)
    plsc.subcore_barrier()

    def body(k, carry):
        base = pl.multiple_of(w * _EPW + k * _CH, 8)
        pltpu.sync_copy(dst_hbm.at[pl.ds(base, _CH)], idx_v)
        pltpu.sync_copy(ones_v, acc.at[idx_v], add=True)
        return carry

    lax.fori_loop(0, _NCHUNK, body, 0)
    plsc.subcore_barrier()
    pltpu.sync_copy(acc.at[pl.ds(s * _RPT, _RPT)],
                    out_hbm.at[c, pl.ds(s * _RPT, _RPT)])


_deg_kernel = pl.kernel(
    _deg_body,
    out_type=jax.ShapeDtypeStruct((_NC, _NPAD), jnp.float32),
    mesh=_sc_mesh,
    scratch_types=[
        pltpu.VMEM((_CH,), jnp.int32),
        pltpu.VMEM((_CH,), jnp.float32),
        pltpu.VMEM_SHARED((_NPAD,), jnp.float32),
        pltpu.SemaphoreType.DMA,
    ],
    compiler_params=_sc_params,
)


def _agg_body(vtab_hbm, src_hbm, dst_hbm, zeros_hbm, out_hbm,
              isrc_v, idst_v, rows_v, acc, sem):
    c = lax.axis_index("c")
    s = lax.axis_index("s")
    w = s * _NC + c
    pltpu.sync_copy(zeros_hbm.at[pl.ds(s * _RPT, _RPT)],
                    acc.at[pl.ds(s * _RPT, _RPT)])
    plsc.subcore_barrier()

    def body(k, carry):
        base = pl.multiple_of(w * _EPW + k * _CH, 8)
        pltpu.sync_copy(src_hbm.at[pl.ds(base, _CH)], isrc_v)
        pltpu.sync_copy(dst_hbm.at[pl.ds(base, _CH)], idst_v)
        pltpu.async_copy(vtab_hbm.at[isrc_v], rows_v, sem).wait()
        pltpu.sync_copy(rows_v, acc.at[idst_v], add=True)
        return carry

    lax.fori_loop(0, _NCHUNK, body, 0)
    plsc.subcore_barrier()
    pltpu.sync_copy(acc.at[pl.ds(s * _RPT, _RPT)],
                    out_hbm.at[c, pl.ds(s * _RPT, _RPT)])


_agg_kernel = pl.kernel(
    _agg_body,
    out_type=jax.ShapeDtypeStruct((_NC, _NPAD, _H), jnp.float32),
    mesh=_sc_mesh,
    scratch_types=[
        pltpu.VMEM((_CH,), jnp.int32),
        pltpu.VMEM((_CH,), jnp.int32),
        pltpu.VMEM((_CH, _H), jnp.float32),
        pltpu.VMEM_SHARED((_NPAD, _H), jnp.float32),
        pltpu.SemaphoreType.DMA,
    ],
    compiler_params=_sc_params,
)


def _tc1_body(x_ref, w1_ref, degp_ref, v1_ref, dinv_ref):
    deg = degp_ref[0] + degp_ref[1] + 1.0
    dinv = lax.rsqrt(deg)
    h0 = jnp.dot(x_ref[...], w1_ref[...], preferred_element_type=jnp.float32)
    v1_ref[...] = h0 * dinv
    dinv_ref[...] = dinv


_tc1 = pl.pallas_call(
    _tc1_body,
    out_shape=[
        jax.ShapeDtypeStruct((_N, _H), jnp.float32),
        jax.ShapeDtypeStruct((_N, 1), jnp.float32),
    ],
)


def _tc2_body(accp_ref, v1_ref, dinv_ref, b1_ref, a_ref, u_ref):
    s = accp_ref[0] + accp_ref[1] + v1_ref[...]
    h1 = s * dinv_ref[...] + b1_ref[...]
    h1p = jnp.where(h1 >= 0.0, h1, a_ref[...] * h1)
    u_ref[...] = h1p * dinv_ref[...]


_tc2 = pl.pallas_call(
    _tc2_body,
    out_shape=jax.ShapeDtypeStruct((_N, _H), jnp.float32),
)


def _tc3_body(accp_ref, u_ref, dinv_ref, batch_ref, w2_ref, b2_ref, out_ref):
    y = (accp_ref[0] + accp_ref[1] + u_ref[...]) * dinv_ref[...]
    gids = lax.broadcasted_iota(jnp.int32, (_G, _N), 0)
    oh = jnp.where(batch_ref[...] == gids, 1.0, 0.0)
    sums = jnp.dot(oh, y, preferred_element_type=jnp.float32)
    cnt = jnp.sum(oh, axis=1, keepdims=True)
    mean = sums / jnp.maximum(cnt, 1.0)
    out_ref[...] = (
        jnp.dot(mean, w2_ref[...], preferred_element_type=jnp.float32)
        + b2_ref[...]
    )


_tc3 = pl.pallas_call(
    _tc3_body,
    out_shape=jax.ShapeDtypeStruct((_G, _O), jnp.float32),
)


def kernel(x, edge_index, batch_idx, W1, b1, prelu_a, W2, b2):
    src = edge_index[0]
    dst = edge_index[1]
    zeros2d = jnp.zeros((_NPAD, _H), jnp.float32)
    zeros1d = jnp.zeros((_NPAD,), jnp.float32)
    ones1d = jnp.ones((_CH,), jnp.float32)

    degp = _deg_kernel(dst, ones1d, zeros1d)
    degp3 = degp[:, :_N, None]
    v1, dinv = _tc1(x, W1, degp3)
    accp1 = _agg_kernel(v1, src, dst, zeros2d)[:, :_N]
    u = _tc2(accp1, v1, dinv, b1.reshape(1, _H), prelu_a.reshape(1, 1))
    accp2 = _agg_kernel(u, src, dst, zeros2d)[:, :_N]
    out = _tc3(accp2, u, dinv, batch_idx.reshape(1, _N), W2,
               b2.reshape(1, _O))
    return out


# speedup vs baseline: 70.3851x; 1.4303x over previous
import jax
import jax.numpy as jnp
from jax import lax
from jax.experimental import pallas as pl
from jax.experimental.pallas import tpu as pltpu
from jax.experimental.pallas import tpu_sc as plsc

_N = 10000
_E = 320000
_D = 128
_H = 16
_G = 16
_O = 2

_NC = 2
_NS = 16
_NW = _NC * _NS
_EPW = _E // _NW
_CH = 1000
_NCHUNK = _EPW // _CH
_NPAD = 10240
_RPT = _NPAD // _NS

_sc_mesh = plsc.VectorSubcoreMesh(core_axis_name="c", subcore_axis_name="s")
_sc_params = pltpu.CompilerParams(use_tc_tiling_on_sc=False)


def _deg_body(edge_hbm, ones_hbm, zeros_hbm, out_hbm,
              idx0, idx1, ones_v, ssem0, ssem1, acc):
    c = lax.axis_index("c")
    s = lax.axis_index("s")
    w = s * _NC + c
    ebase = w * _EPW
    idx = [idx0, idx1]
    ssem = [ssem0, ssem1]

    pltpu.sync_copy(zeros_hbm.at[pl.ds(s * _RPT, _RPT)],
                    acc.at[pl.ds(s * _RPT, _RPT)])
    pltpu.sync_copy(ones_hbm.at[pl.ds(0, _CH)], ones_v)
    plsc.subcore_barrier()

    scat = [None, None]
    pltpu.sync_copy(edge_hbm.at[1, pl.ds(ebase, _CH)], idx[0])
    for k in range(_NCHUNK):
        cur = k % 2
        nxt = 1 - cur
        if k + 1 < _NCHUNK:
            if scat[nxt] is not None:
                scat[nxt].wait()
            pltpu.sync_copy(edge_hbm.at[1, pl.ds(ebase + (k + 1) * _CH, _CH)],
                            idx[nxt])
        scat[cur] = pltpu.async_copy(ones_v, acc.at[idx[cur]],
                                     ssem[cur], add=True)
    for d in scat:
        if d is not None:
            d.wait()
    plsc.subcore_barrier()
    pltpu.sync_copy(acc.at[pl.ds(s * _RPT, _RPT)],
                    out_hbm.at[c, pl.ds(s * _RPT, _RPT)])


_deg_kernel = pl.kernel(
    _deg_body,
    out_type=jax.ShapeDtypeStruct((_NC, _NPAD), jnp.float32),
    mesh=_sc_mesh,
    scratch_types=[
        pltpu.VMEM((_CH,), jnp.int32),
        pltpu.VMEM((_CH,), jnp.int32),
        pltpu.VMEM((_CH,), jnp.float32),
        pltpu.SemaphoreType.DMA,
        pltpu.SemaphoreType.DMA,
        pltpu.VMEM_SHARED((_NPAD,), jnp.float32),
    ],
    compiler_params=_sc_params,
)


def _agg_body(vtab_hbm, edge_hbm, zeros_hbm, out_hbm,
              isrc0, isrc1, idst0, idst1, rows0, rows1,
              gsem0, gsem1, ssem0, ssem1, acc):
    c = lax.axis_index("c")
    s = lax.axis_index("s")
    w = s * _NC + c
    ebase = w * _EPW
    isrc = [isrc0, isrc1]
    idst = [idst0, idst1]
    rows = [rows0, rows1]
    gsem = [gsem0, gsem1]
    ssem = [ssem0, ssem1]

    pltpu.sync_copy(zeros_hbm.at[pl.ds(s * _RPT, _RPT)],
                    acc.at[pl.ds(s * _RPT, _RPT)])
    plsc.subcore_barrier()

    scat = [None, None]
    pltpu.sync_copy(edge_hbm.at[0, pl.ds(ebase, _CH)], isrc[0])
    pltpu.sync_copy(edge_hbm.at[1, pl.ds(ebase, _CH)], idst[0])
    gat = pltpu.async_copy(vtab_hbm.at[isrc[0]], rows[0], gsem[0])
    for k in range(_NCHUNK):
        cur = k % 2
        nxt = 1 - cur
        ngat = None
        if k + 1 < _NCHUNK:
            if scat[nxt] is not None:
                scat[nxt].wait()
            base = ebase + (k + 1) * _CH
            pltpu.sync_copy(edge_hbm.at[0, pl.ds(base, _CH)], isrc[nxt])
            pltpu.sync_copy(edge_hbm.at[1, pl.ds(base, _CH)], idst[nxt])
            ngat = pltpu.async_copy(vtab_hbm.at[isrc[nxt]], rows[nxt],
                                    gsem[nxt])
        gat.wait()
        scat[cur] = pltpu.async_copy(rows[cur], acc.at[idst[cur]],
                                     ssem[cur], add=True)
        gat = ngat
    for d in scat:
        if d is not None:
            d.wait()

    plsc.subcore_barrier()
    pltpu.sync_copy(acc.at[pl.ds(s * _RPT, _RPT)],
                    out_hbm.at[c, pl.ds(s * _RPT, _RPT)])


_agg_kernel = pl.kernel(
    _agg_body,
    out_type=jax.ShapeDtypeStruct((_NC, _NPAD, _H), jnp.float32),
    mesh=_sc_mesh,
    scratch_types=[
        pltpu.VMEM((_CH,), jnp.int32),
        pltpu.VMEM((_CH,), jnp.int32),
        pltpu.VMEM((_CH,), jnp.int32),
        pltpu.VMEM((_CH,), jnp.int32),
        pltpu.VMEM((_CH, _H), jnp.float32),
        pltpu.VMEM((_CH, _H), jnp.float32),
        pltpu.SemaphoreType.DMA,
        pltpu.SemaphoreType.DMA,
        pltpu.SemaphoreType.DMA,
        pltpu.SemaphoreType.DMA,
        pltpu.VMEM_SHARED((_NPAD, _H), jnp.float32),
    ],
    compiler_params=_sc_params,
)


def _tc1a_body(x_ref, w1_ref, h0_ref):
    h0_ref[...] = jnp.dot(x_ref[...], w1_ref[...],
                          preferred_element_type=jnp.float32)


_tc1a = pl.pallas_call(
    _tc1a_body,
    out_shape=jax.ShapeDtypeStruct((_N, _H), jnp.float32),
)


def _tc1b_body(degp_ref, h0_ref, v1_ref, dinv_ref):
    deg = degp_ref[0, :_N] + degp_ref[1, :_N] + 1.0
    dinv = lax.rsqrt(deg).reshape(_N, 1)
    v1_ref[...] = h0_ref[...] * dinv
    dinv_ref[...] = dinv


_tc1b = pl.pallas_call(
    _tc1b_body,
    out_shape=[
        jax.ShapeDtypeStruct((_N, _H), jnp.float32),
        jax.ShapeDtypeStruct((_N, 1), jnp.float32),
    ],
)


def _tc2_body(accp_ref, v1_ref, dinv_ref, b1_ref, a_ref, u_ref):
    s = accp_ref[0, :_N, :] + accp_ref[1, :_N, :] + v1_ref[...]
    h1 = s * dinv_ref[...] + b1_ref[...].reshape(1, _H)
    h1p = jnp.where(h1 >= 0.0, h1, a_ref[0] * h1)
    u_ref[...] = h1p * dinv_ref[...]


_tc2 = pl.pallas_call(
    _tc2_body,
    out_shape=jax.ShapeDtypeStruct((_N, _H), jnp.float32),
)


def _tc3_body(accp_ref, u_ref, dinv_ref, batch_ref, w2_ref, b2_ref, out_ref):
    y = (accp_ref[0, :_N, :] + accp_ref[1, :_N, :] + u_ref[...]) \
        * dinv_ref[...]
    gids = lax.broadcasted_iota(jnp.int32, (_G, _N), 0)
    oh = jnp.where(batch_ref[...].reshape(1, _N) == gids, 1.0, 0.0)
    sums = jnp.dot(oh, y, preferred_element_type=jnp.float32)
    cnt = jnp.sum(oh, axis=1, keepdims=True)
    mean = sums / jnp.maximum(cnt, 1.0)
    out_ref[...] = (
        jnp.dot(mean, w2_ref[...], preferred_element_type=jnp.float32)
        + b2_ref[...].reshape(1, _O)
    )


_tc3 = pl.pallas_call(
    _tc3_body,
    out_shape=jax.ShapeDtypeStruct((_G, _O), jnp.float32),
)


def kernel(x, edge_index, batch_idx, W1, b1, prelu_a, W2, b2):
    zeros2d = jnp.zeros((_NPAD, _H), jnp.float32)
    zeros1d = jnp.zeros((_NPAD,), jnp.float32)
    ones1d = jnp.ones((_CH,), jnp.float32)

    degp = _deg_kernel(edge_index, ones1d, zeros1d)
    h0 = _tc1a(x, W1)
    v1, dinv = _tc1b(degp, h0)
    accp1 = _agg_kernel(v1, edge_index, zeros2d)
    u = _tc2(accp1, v1, dinv, b1, prelu_a)
    accp2 = _agg_kernel(u, edge_index, zeros2d)
    out = _tc3(accp2, u, dinv, batch_idx, W2, b2)
    return out


# speedup vs baseline: 94.1839x; 1.3381x over previous
import jax
import jax.numpy as jnp
from jax import lax
from jax.experimental import pallas as pl
from jax.experimental.pallas import tpu as pltpu
from jax.experimental.pallas import tpu_sc as plsc

_N = 10000
_E = 320000
_D = 128
_H = 16
_G = 16
_O = 2

_NC = 2
_NS = 16
_NW = _NC * _NS
_EPW = _E // _NW
_CH = 1000
_NCHUNK = _EPW // _CH
_NPAD = 10240
_RPT = _NPAD // _NS
_PR = _NPAD * _H // 128

_sc_mesh = plsc.VectorSubcoreMesh(core_axis_name="c", subcore_axis_name="s")
_sc_params = pltpu.CompilerParams(use_tc_tiling_on_sc=False)


def _deg_body(edge_hbm, ones_hbm, zeros_hbm, out_hbm,
              idx0, idx1, ones_v, ssem0, ssem1, acc):
    c = lax.axis_index("c")
    s = lax.axis_index("s")
    w = s * _NC + c
    ebase = w * _EPW
    idx = [idx0, idx1]
    ssem = [ssem0, ssem1]

    pltpu.sync_copy(zeros_hbm.at[pl.ds(s * _RPT, _RPT)],
                    acc.at[pl.ds(s * _RPT, _RPT)])
    pltpu.sync_copy(ones_hbm.at[pl.ds(0, _CH)], ones_v)
    plsc.subcore_barrier()

    scat = [None, None]
    pltpu.sync_copy(edge_hbm.at[1, pl.ds(ebase, _CH)], idx[0])
    for k in range(_NCHUNK):
        cur = k % 2
        nxt = 1 - cur
        if k + 1 < _NCHUNK:
            if scat[nxt] is not None:
                scat[nxt].wait()
            pltpu.sync_copy(edge_hbm.at[1, pl.ds(ebase + (k + 1) * _CH, _CH)],
                            idx[nxt])
        scat[cur] = pltpu.async_copy(ones_v, acc.at[idx[cur]],
                                     ssem[cur], add=True)
    for d in scat:
        if d is not None:
            d.wait()
    plsc.subcore_barrier()
    pltpu.sync_copy(acc.at[pl.ds(s * _RPT, _RPT)],
                    out_hbm.at[c, pl.ds(s * _RPT, _RPT)])


_deg_kernel = pl.kernel(
    _deg_body,
    out_type=jax.ShapeDtypeStruct((_NC, _NPAD, _H), jnp.float32),
    mesh=_sc_mesh,
    scratch_types=[
        pltpu.VMEM((_CH,), jnp.int32),
        pltpu.VMEM((_CH,), jnp.int32),
        pltpu.VMEM((_CH, _H), jnp.float32),
        pltpu.SemaphoreType.DMA,
        pltpu.SemaphoreType.DMA,
        pltpu.VMEM_SHARED((_NPAD, _H), jnp.float32),
    ],
    compiler_params=_sc_params,
)


def _agg_body(vtab_hbm, edge_hbm, zeros_hbm, out_hbm,
              isrc0, isrc1, idst0, idst1, rows0, rows1,
              gsem0, gsem1, ssem0, ssem1, acc):
    c = lax.axis_index("c")
    s = lax.axis_index("s")
    w = s * _NC + c
    ebase = w * _EPW
    isrc = [isrc0, isrc1]
    idst = [idst0, idst1]
    rows = [rows0, rows1]
    gsem = [gsem0, gsem1]
    ssem = [ssem0, ssem1]

    pltpu.sync_copy(zeros_hbm.at[pl.ds(s * _RPT, _RPT)],
                    acc.at[pl.ds(s * _RPT, _RPT)])
    plsc.subcore_barrier()

    scat = [None, None]
    pltpu.sync_copy(edge_hbm.at[0, pl.ds(ebase, _CH)], isrc[0])
    pltpu.sync_copy(edge_hbm.at[1, pl.ds(ebase, _CH)], idst[0])
    gat = pltpu.async_copy(vtab_hbm.at[isrc[0]], rows[0], gsem[0])
    for k in range(_NCHUNK):
        cur = k % 2
        nxt = 1 - cur
        ngat = None
        if k + 1 < _NCHUNK:
            if scat[nxt] is not None:
                scat[nxt].wait()
            base = ebase + (k + 1) * _CH
            pltpu.sync_copy(edge_hbm.at[0, pl.ds(base, _CH)], isrc[nxt])
            pltpu.sync_copy(edge_hbm.at[1, pl.ds(base, _CH)], idst[nxt])
            ngat = pltpu.async_copy(vtab_hbm.at[isrc[nxt]], rows[nxt],
                                    gsem[nxt])
        gat.wait()
        scat[cur] = pltpu.async_copy(rows[cur], acc.at[idst[cur]],
                                     ssem[cur], add=True)
        gat = ngat
    for d in scat:
        if d is not None:
            d.wait()

    plsc.subcore_barrier()
    pltpu.sync_copy(acc.at[pl.ds(s * _RPT, _RPT)],
                    out_hbm.at[c, pl.ds(s * _RPT, _RPT)])


_agg_kernel = pl.kernel(
    _agg_body,
    out_type=jax.ShapeDtypeStruct((_NC, _NPAD, _H), jnp.float32),
    mesh=_sc_mesh,
    scratch_types=[
        pltpu.VMEM((_CH,), jnp.int32),
        pltpu.VMEM((_CH,), jnp.int32),
        pltpu.VMEM((_CH,), jnp.int32),
        pltpu.VMEM((_CH,), jnp.int32),
        pltpu.VMEM((_CH, _H), jnp.float32),
        pltpu.VMEM((_CH, _H), jnp.float32),
        pltpu.SemaphoreType.DMA,
        pltpu.SemaphoreType.DMA,
        pltpu.SemaphoreType.DMA,
        pltpu.SemaphoreType.DMA,
        pltpu.VMEM_SHARED((_NPAD, _H), jnp.float32),
    ],
    compiler_params=_sc_params,
)


def _tc1a_body(x_ref, w1_ref, h0_ref):
    h = jnp.dot(x_ref[...], w1_ref[...], preferred_element_type=jnp.float32)
    h0_ref[...] = jnp.concatenate(
        [h, jnp.zeros((_NPAD - _N, _H), jnp.float32)], axis=0)


_tc1a = pl.pallas_call(
    _tc1a_body,
    out_shape=jax.ShapeDtypeStruct((_NPAD, _H), jnp.float32),
)


def _tc1b_body(degp_ref, h0_ref, v1_ref, dinv_ref):
    deg = degp_ref[0] + degp_ref[1] + 1.0
    dinv = lax.rsqrt(deg)
    v1_ref[...] = h0_ref[...] * dinv
    dinv_ref[...] = dinv


_tc1b = pl.pallas_call(
    _tc1b_body,
    out_shape=[
        jax.ShapeDtypeStruct((_PR, 128), jnp.float32),
        jax.ShapeDtypeStruct((_PR, 128), jnp.float32),
    ],
)


def _tc2_body(accp_ref, v1_ref, dinv_ref, b1_ref, a_ref, u_ref):
    dinv = dinv_ref[...]
    s = accp_ref[0] + accp_ref[1] + v1_ref[...]
    h1 = s * dinv + b1_ref[...].reshape(1, 128)
    h1p = jnp.where(h1 >= 0.0, h1, a_ref[0] * h1)
    u_ref[...] = h1p * dinv


_tc2 = pl.pallas_call(
    _tc2_body,
    out_shape=jax.ShapeDtypeStruct((_PR, 128), jnp.float32),
)


def _tc3_body(accp_ref, u_ref, dinv_ref, bbp_ref, w2_ref, b2_ref, out_ref):
    yp = (accp_ref[0] + accp_ref[1] + u_ref[...]) * dinv_ref[...]
    bb = bbp_ref[...]
    mrows = []
    for g in range(_G):
        m = bb == g
        s128 = jnp.sum(jnp.where(m, yp, 0.0), axis=0, keepdims=True)
        c128 = jnp.sum(jnp.where(m, 1.0, 0.0), axis=0, keepdims=True)
        s16 = s128[:, 0:_H]
        c16 = c128[:, 0:_H]
        for j in range(1, 128 // _H):
            s16 = s16 + s128[:, _H * j:_H * (j + 1)]
            c16 = c16 + c128[:, _H * j:_H * (j + 1)]
        mrows.append(s16 / jnp.maximum(c16, 1.0))
    mean = jnp.concatenate(mrows, axis=0)
    out_ref[...] = (
        jnp.dot(mean, w2_ref[...], preferred_element_type=jnp.float32)
        + b2_ref[...].reshape(1, _O)
    )


_tc3 = pl.pallas_call(
    _tc3_body,
    out_shape=jax.ShapeDtypeStruct((_G, _O), jnp.float32),
)


def kernel(x, edge_index, batch_idx, W1, b1, prelu_a, W2, b2):
    zeros2d = jnp.zeros((_NPAD, _H), jnp.float32)
    ones2d = jnp.ones((_CH, _H), jnp.float32)

    deg16 = _deg_kernel(edge_index, ones2d, zeros2d)
    deg16p = deg16.reshape(_NC, _PR, 128)
    h0 = _tc1a(x, W1)
    h0p = h0.reshape(_PR, 128)
    v1p, dinv16p = _tc1b(deg16p, h0p)
    v1 = v1p.reshape(_NPAD, _H)
    accp1 = _agg_kernel(v1, edge_index, zeros2d).reshape(_NC, _PR, 128)
    up = _tc2(accp1, v1p, dinv16p, jnp.tile(b1, 128 // _H), prelu_a)
    u = up.reshape(_NPAD, _H)
    accp2 = _agg_kernel(u, edge_index, zeros2d).reshape(_NC, _PR, 128)
    bpad = jnp.concatenate(
        [batch_idx, jnp.full((_NPAD - _N,), -1, jnp.int32)])
    bbp = jnp.broadcast_to(bpad[:, None], (_NPAD, _H)).reshape(_PR, 128)
    out = _tc3(accp2, up, dinv16p, bbp, W2, b2)
    return out


# speedup vs baseline: 97.8069x; 1.0385x over previous
import jax
import jax.numpy as jnp
from jax import lax
from jax.experimental import pallas as pl
from jax.experimental.pallas import tpu as pltpu
from jax.experimental.pallas import tpu_sc as plsc

_N = 10000
_E = 320000
_D = 128
_H = 16
_G = 16
_O = 2

_NC = 2
_NS = 16
_NW = _NC * _NS
_EPW = _E // _NW
_CH = 2000
_NCHUNK = _EPW // _CH
_NPAD = 10240
_RPT = _NPAD // _NS
_PR = _NPAD * _H // 128

_sc_mesh = plsc.VectorSubcoreMesh(core_axis_name="c", subcore_axis_name="s")
_sc_params = pltpu.CompilerParams(use_tc_tiling_on_sc=False)


def _deg_body(edge_hbm, ones_hbm, zeros_hbm, out_hbm,
              idx0, idx1, ones_v, ssem0, ssem1, acc):
    c = lax.axis_index("c")
    s = lax.axis_index("s")
    w = s * _NC + c
    ebase = w * _EPW
    idx = [idx0, idx1]
    ssem = [ssem0, ssem1]

    pltpu.sync_copy(zeros_hbm.at[pl.ds(s * _RPT, _RPT)],
                    acc.at[pl.ds(s * _RPT, _RPT)])
    pltpu.sync_copy(ones_hbm.at[pl.ds(0, _CH)], ones_v)
    plsc.subcore_barrier()

    scat = [None, None]
    pltpu.sync_copy(edge_hbm.at[1, pl.ds(ebase, _CH)], idx[0])
    for k in range(_NCHUNK):
        cur = k % 2
        nxt = 1 - cur
        if k + 1 < _NCHUNK:
            if scat[nxt] is not None:
                scat[nxt].wait()
            pltpu.sync_copy(edge_hbm.at[1, pl.ds(ebase + (k + 1) * _CH, _CH)],
                            idx[nxt])
        scat[cur] = pltpu.async_copy(ones_v, acc.at[idx[cur]],
                                     ssem[cur], add=True)
    for d in scat:
        if d is not None:
            d.wait()
    plsc.subcore_barrier()
    pltpu.sync_copy(acc.at[pl.ds(s * _RPT, _RPT)],
                    out_hbm.at[c, pl.ds(s * _RPT, _RPT)])


_deg_kernel = pl.kernel(
    _deg_body,
    out_type=jax.ShapeDtypeStruct((_NC, _NPAD, _H), jnp.float32),
    mesh=_sc_mesh,
    scratch_types=[
        pltpu.VMEM((_CH,), jnp.int32),
        pltpu.VMEM((_CH,), jnp.int32),
        pltpu.VMEM((_CH, _H), jnp.float32),
        pltpu.SemaphoreType.DMA,
        pltpu.SemaphoreType.DMA,
        pltpu.VMEM_SHARED((_NPAD, _H), jnp.float32),
    ],
    compiler_params=_sc_params,
)


def _agg_body(vtab_hbm, edge_hbm, zeros_hbm, out_hbm,
              isrc0, isrc1, idst0, idst1, rows0, rows1,
              gsem0, gsem1, ssem0, ssem1, acc):
    c = lax.axis_index("c")
    s = lax.axis_index("s")
    w = s * _NC + c
    ebase = w * _EPW
    isrc = [isrc0, isrc1]
    idst = [idst0, idst1]
    rows = [rows0, rows1]
    gsem = [gsem0, gsem1]
    ssem = [ssem0, ssem1]

    pltpu.sync_copy(zeros_hbm.at[pl.ds(s * _RPT, _RPT)],
                    acc.at[pl.ds(s * _RPT, _RPT)])
    plsc.subcore_barrier()

    scat = [None, None]
    pltpu.sync_copy(edge_hbm.at[0, pl.ds(ebase, _CH)], isrc[0])
    pltpu.sync_copy(edge_hbm.at[1, pl.ds(ebase, _CH)], idst[0])
    gat = pltpu.async_copy(vtab_hbm.at[isrc[0]], rows[0], gsem[0])
    for k in range(_NCHUNK):
        cur = k % 2
        nxt = 1 - cur
        ngat = None
        if k + 1 < _NCHUNK:
            if scat[nxt] is not None:
                scat[nxt].wait()
            base = ebase + (k + 1) * _CH
            pltpu.sync_copy(edge_hbm.at[0, pl.ds(base, _CH)], isrc[nxt])
            pltpu.sync_copy(edge_hbm.at[1, pl.ds(base, _CH)], idst[nxt])
            ngat = pltpu.async_copy(vtab_hbm.at[isrc[nxt]], rows[nxt],
                                    gsem[nxt])
        gat.wait()
        scat[cur] = pltpu.async_copy(rows[cur], acc.at[idst[cur]],
                                     ssem[cur], add=True)
        gat = ngat
    for d in scat:
        if d is not None:
            d.wait()

    plsc.subcore_barrier()
    pltpu.sync_copy(acc.at[pl.ds(s * _RPT, _RPT)],
                    out_hbm.at[c, pl.ds(s * _RPT, _RPT)])


_agg_kernel = pl.kernel(
    _agg_body,
    out_type=jax.ShapeDtypeStruct((_NC, _NPAD, _H), jnp.float32),
    mesh=_sc_mesh,
    scratch_types=[
        pltpu.VMEM((_CH,), jnp.int32),
        pltpu.VMEM((_CH,), jnp.int32),
        pltpu.VMEM((_CH,), jnp.int32),
        pltpu.VMEM((_CH,), jnp.int32),
        pltpu.VMEM((_CH, _H), jnp.float32),
        pltpu.VMEM((_CH, _H), jnp.float32),
        pltpu.SemaphoreType.DMA,
        pltpu.SemaphoreType.DMA,
        pltpu.SemaphoreType.DMA,
        pltpu.SemaphoreType.DMA,
        pltpu.VMEM_SHARED((_NPAD, _H), jnp.float32),
    ],
    compiler_params=_sc_params,
)


def _tc1a_body(x_ref, w1_ref, h0_ref):
    h = jnp.dot(x_ref[...], w1_ref[...], preferred_element_type=jnp.float32)
    h0_ref[...] = jnp.concatenate(
        [h, jnp.zeros((_NPAD - _N, _H), jnp.float32)], axis=0)


_tc1a = pl.pallas_call(
    _tc1a_body,
    out_shape=jax.ShapeDtypeStruct((_NPAD, _H), jnp.float32),
)


def _tc1b_body(degp_ref, h0_ref, v1_ref, dinv_ref):
    deg = degp_ref[0] + degp_ref[1] + 1.0
    dinv = lax.rsqrt(deg)
    v1_ref[...] = h0_ref[...] * dinv
    dinv_ref[...] = dinv


_tc1b = pl.pallas_call(
    _tc1b_body,
    out_shape=[
        jax.ShapeDtypeStruct((_PR, 128), jnp.float32),
        jax.ShapeDtypeStruct((_PR, 128), jnp.float32),
    ],
)


def _tc2_body(accp_ref, v1_ref, dinv_ref, b1_ref, a_ref, u_ref):
    dinv = dinv_ref[...]
    s = accp_ref[0] + accp_ref[1] + v1_ref[...]
    h1 = s * dinv + b1_ref[...].reshape(1, 128)
    h1p = jnp.where(h1 >= 0.0, h1, a_ref[0] * h1)
    u_ref[...] = h1p * dinv


_tc2 = pl.pallas_call(
    _tc2_body,
    out_shape=jax.ShapeDtypeStruct((_PR, 128), jnp.float32),
)


def _tc3_body(accp_ref, u_ref, dinv_ref, bbp_ref, w2_ref, b2_ref, out_ref):
    yp = (accp_ref[0] + accp_ref[1] + u_ref[...]) * dinv_ref[...]
    bb = bbp_ref[...]
    mrows = []
    for g in range(_G):
        m = bb == g
        s128 = jnp.sum(jnp.where(m, yp, 0.0), axis=0, keepdims=True)
        c128 = jnp.sum(jnp.where(m, 1.0, 0.0), axis=0, keepdims=True)
        s16 = s128[:, 0:_H]
        c16 = c128[:, 0:_H]
        for j in range(1, 128 // _H):
            s16 = s16 + s128[:, _H * j:_H * (j + 1)]
            c16 = c16 + c128[:, _H * j:_H * (j + 1)]
        mrows.append(s16 / jnp.maximum(c16, 1.0))
    mean = jnp.concatenate(mrows, axis=0)
    out_ref[...] = (
        jnp.dot(mean, w2_ref[...], preferred_element_type=jnp.float32)
        + b2_ref[...].reshape(1, _O)
    )


_tc3 = pl.pallas_call(
    _tc3_body,
    out_shape=jax.ShapeDtypeStruct((_G, _O), jnp.float32),
)


def kernel(x, edge_index, batch_idx, W1, b1, prelu_a, W2, b2):
    zeros2d = jnp.zeros((_NPAD, _H), jnp.float32)
    ones2d = jnp.ones((_CH, _H), jnp.float32)

    deg16 = _deg_kernel(edge_index, ones2d, zeros2d)
    deg16p = deg16.reshape(_NC, _PR, 128)
    h0 = _tc1a(x, W1)
    h0p = h0.reshape(_PR, 128)
    v1p, dinv16p = _tc1b(deg16p, h0p)
    v1 = v1p.reshape(_NPAD, _H)
    accp1 = _agg_kernel(v1, edge_index, zeros2d).reshape(_NC, _PR, 128)
    up = _tc2(accp1, v1p, dinv16p, jnp.tile(b1, 128 // _H), prelu_a)
    u = up.reshape(_NPAD, _H)
    accp2 = _agg_kernel(u, edge_index, zeros2d).reshape(_NC, _PR, 128)
    bpad = jnp.concatenate(
        [batch_idx, jnp.full((_NPAD - _N,), -1, jnp.int32)])
    bbp = jnp.broadcast_to(bpad[:, None], (_NPAD, _H)).reshape(_PR, 128)
    out = _tc3(accp2, up, dinv16p, bbp, W2, b2)
    return out


# speedup vs baseline: 99.5371x; 1.0177x over previous
import jax
import jax.numpy as jnp
from jax import lax
from jax.experimental import pallas as pl
from jax.experimental.pallas import tpu as pltpu
from jax.experimental.pallas import tpu_sc as plsc

_N = 10000
_E = 320000
_D = 128
_H = 16
_G = 16
_O = 2

_NC = 2
_NS = 16
_NW = _NC * _NS
_EPW = _E // _NW
_CH = 2000
_NCHUNK = _EPW // _CH
_CHD = 1000
_NCHUNKD = _EPW // _CHD
_NPAD = 10240
_RPT = _NPAD // _NS
_PR = _NPAD * _H // 128

_sc_mesh = plsc.VectorSubcoreMesh(core_axis_name="c", subcore_axis_name="s")
_sc_params = pltpu.CompilerParams(use_tc_tiling_on_sc=False)


def _deg_body(edge_hbm, ones_hbm, zeros_hbm, out_hbm,
              idx0, idx1, ones_v, ssem0, ssem1, acc):
    c = lax.axis_index("c")
    s = lax.axis_index("s")
    w = s * _NC + c
    ebase = w * _EPW
    idx = [idx0, idx1]
    ssem = [ssem0, ssem1]

    pltpu.sync_copy(zeros_hbm.at[pl.ds(s * _RPT, _RPT)],
                    acc.at[pl.ds(s * _RPT, _RPT)])
    pltpu.sync_copy(ones_hbm.at[pl.ds(0, _CHD)], ones_v)
    plsc.subcore_barrier()

    scat = [None, None]
    pltpu.sync_copy(edge_hbm.at[1, pl.ds(ebase, _CHD)], idx[0])
    for k in range(_NCHUNKD):
        cur = k % 2
        nxt = 1 - cur
        if k + 1 < _NCHUNKD:
            if scat[nxt] is not None:
                scat[nxt].wait()
            pltpu.sync_copy(edge_hbm.at[1, pl.ds(ebase + (k + 1) * _CHD, _CHD)],
                            idx[nxt])
        scat[cur] = pltpu.async_copy(ones_v, acc.at[idx[cur]],
                                     ssem[cur], add=True)
    for d in scat:
        if d is not None:
            d.wait()
    plsc.subcore_barrier()
    pltpu.sync_copy(acc.at[pl.ds(s * _RPT, _RPT)],
                    out_hbm.at[c, pl.ds(s * _RPT, _RPT)])


_deg_kernel = pl.kernel(
    _deg_body,
    out_type=jax.ShapeDtypeStruct((_NC, _NPAD, _H), jnp.float32),
    mesh=_sc_mesh,
    scratch_types=[
        pltpu.VMEM((_CHD,), jnp.int32),
        pltpu.VMEM((_CHD,), jnp.int32),
        pltpu.VMEM((_CHD, _H), jnp.float32),
        pltpu.SemaphoreType.DMA,
        pltpu.SemaphoreType.DMA,
        pltpu.VMEM_SHARED((_NPAD, _H), jnp.float32),
    ],
    compiler_params=_sc_params,
)


def _agg_body(vtab_hbm, edge_hbm, zeros_hbm, out_hbm,
              isrc0, isrc1, idst0, idst1, rows0, rows1,
              gsem0, gsem1, ssem0, ssem1, acc):
    c = lax.axis_index("c")
    s = lax.axis_index("s")
    w = s * _NC + c
    ebase = w * _EPW
    isrc = [isrc0, isrc1]
    idst = [idst0, idst1]
    rows = [rows0, rows1]
    gsem = [gsem0, gsem1]
    ssem = [ssem0, ssem1]

    pltpu.sync_copy(zeros_hbm.at[pl.ds(s * _RPT, _RPT)],
                    acc.at[pl.ds(s * _RPT, _RPT)])
    plsc.subcore_barrier()

    scat = [None, None]
    pltpu.sync_copy(edge_hbm.at[0, pl.ds(ebase, _CH)], isrc[0])
    pltpu.sync_copy(edge_hbm.at[1, pl.ds(ebase, _CH)], idst[0])
    gat = pltpu.async_copy(vtab_hbm.at[isrc[0]], rows[0], gsem[0])
    for k in range(_NCHUNK):
        cur = k % 2
        nxt = 1 - cur
        ngat = None
        if k + 1 < _NCHUNK:
            if scat[nxt] is not None:
                scat[nxt].wait()
            base = ebase + (k + 1) * _CH
            pltpu.sync_copy(edge_hbm.at[0, pl.ds(base, _CH)], isrc[nxt])
            pltpu.sync_copy(edge_hbm.at[1, pl.ds(base, _CH)], idst[nxt])
            ngat = pltpu.async_copy(vtab_hbm.at[isrc[nxt]], rows[nxt],
                                    gsem[nxt])
        gat.wait()
        scat[cur] = pltpu.async_copy(rows[cur], acc.at[idst[cur]],
                                     ssem[cur], add=True)
        gat = ngat
    for d in scat:
        if d is not None:
            d.wait()

    plsc.subcore_barrier()
    pltpu.sync_copy(acc.at[pl.ds(s * _RPT, _RPT)],
                    out_hbm.at[c, pl.ds(s * _RPT, _RPT)])


_agg_kernel = pl.kernel(
    _agg_body,
    out_type=jax.ShapeDtypeStruct((_NC, _NPAD, _H), jnp.float32),
    mesh=_sc_mesh,
    scratch_types=[
        pltpu.VMEM((_CH,), jnp.int32),
        pltpu.VMEM((_CH,), jnp.int32),
        pltpu.VMEM((_CH,), jnp.int32),
        pltpu.VMEM((_CH,), jnp.int32),
        pltpu.VMEM((_CH, _H), jnp.float32),
        pltpu.VMEM((_CH, _H), jnp.float32),
        pltpu.SemaphoreType.DMA,
        pltpu.SemaphoreType.DMA,
        pltpu.SemaphoreType.DMA,
        pltpu.SemaphoreType.DMA,
        pltpu.VMEM_SHARED((_NPAD, _H), jnp.float32),
    ],
    compiler_params=_sc_params,
)


def _tc1a_body(x_ref, w1_ref, h0_ref):
    h = jnp.dot(x_ref[...], w1_ref[...], preferred_element_type=jnp.float32)
    h0_ref[...] = jnp.concatenate(
        [h, jnp.zeros((_NPAD - _N, _H), jnp.float32)], axis=0)


_tc1a = pl.pallas_call(
    _tc1a_body,
    out_shape=jax.ShapeDtypeStruct((_NPAD, _H), jnp.float32),
)


def _tc1b_body(degp_ref, h0_ref, v1_ref, dinv_ref):
    deg = degp_ref[0] + degp_ref[1] + 1.0
    dinv = lax.rsqrt(deg)
    v1_ref[...] = h0_ref[...] * dinv
    dinv_ref[...] = dinv


_tc1b = pl.pallas_call(
    _tc1b_body,
    out_shape=[
        jax.ShapeDtypeStruct((_PR, 128), jnp.float32),
        jax.ShapeDtypeStruct((_PR, 128), jnp.float32),
    ],
)


def _tc2_body(accp_ref, v1_ref, dinv_ref, b1_ref, a_ref, u_ref):
    dinv = dinv_ref[...]
    s = accp_ref[0] + accp_ref[1] + v1_ref[...]
    h1 = s * dinv + b1_ref[...].reshape(1, 128)
    h1p = jnp.where(h1 >= 0.0, h1, a_ref[0] * h1)
    u_ref[...] = h1p * dinv


_tc2 = pl.pallas_call(
    _tc2_body,
    out_shape=jax.ShapeDtypeStruct((_PR, 128), jnp.float32),
)


def _tc3_body(accp_ref, u_ref, dinv_ref, bbp_ref, w2_ref, b2_ref, out_ref):
    yp = (accp_ref[0] + accp_ref[1] + u_ref[...]) * dinv_ref[...]
    bb = bbp_ref[...]
    mrows = []
    for g in range(_G):
        m = bb == g
        s128 = jnp.sum(jnp.where(m, yp, 0.0), axis=0, keepdims=True)
        c128 = jnp.sum(jnp.where(m, 1.0, 0.0), axis=0, keepdims=True)
        s16 = s128[:, 0:_H]
        c16 = c128[:, 0:_H]
        for j in range(1, 128 // _H):
            s16 = s16 + s128[:, _H * j:_H * (j + 1)]
            c16 = c16 + c128[:, _H * j:_H * (j + 1)]
        mrows.append(s16 / jnp.maximum(c16, 1.0))
    mean = jnp.concatenate(mrows, axis=0)
    out_ref[...] = (
        jnp.dot(mean, w2_ref[...], preferred_element_type=jnp.float32)
        + b2_ref[...].reshape(1, _O)
    )


_tc3 = pl.pallas_call(
    _tc3_body,
    out_shape=jax.ShapeDtypeStruct((_G, _O), jnp.float32),
)


def kernel(x, edge_index, batch_idx, W1, b1, prelu_a, W2, b2):
    zeros2d = jnp.zeros((_NPAD, _H), jnp.float32)
    ones2d = jnp.ones((_CHD, _H), jnp.float32)

    deg16 = _deg_kernel(edge_index, ones2d, zeros2d)
    deg16p = deg16.reshape(_NC, _PR, 128)
    h0 = _tc1a(x, W1)
    h0p = h0.reshape(_PR, 128)
    v1p, dinv16p = _tc1b(deg16p, h0p)
    v1 = v1p.reshape(_NPAD, _H)
    accp1 = _agg_kernel(v1, edge_index, zeros2d).reshape(_NC, _PR, 128)
    up = _tc2(accp1, v1p, dinv16p, jnp.tile(b1, 128 // _H), prelu_a)
    u = up.reshape(_NPAD, _H)
    accp2 = _agg_kernel(u, edge_index, zeros2d).reshape(_NC, _PR, 128)
    bpad = jnp.concatenate(
        [batch_idx, jnp.full((_NPAD - _N,), -1, jnp.int32)])
    bbp = jnp.broadcast_to(bpad[:, None], (_NPAD, _H)).reshape(_PR, 128)
    out = _tc3(accp2, up, dinv16p, bbp, W2, b2)
    return out
